# Initial kernel scaffold; baseline (speedup 1.0000x reference)
import jax
import jax.numpy as jnp
from jax.experimental import pallas as pl


def kernel(logits):
    raise NotImplementedError("write your pallas kernel here")


# speedup vs baseline: 19.5556x; 19.5556x over previous
import jax
import jax.numpy as jnp
from jax.experimental import pallas as pl

_K = 2048
_ROWS_PER_BLOCK = 32


def _mask_body(x_ref, o_ref):
    x = x_ref[...]
    xi = jax.lax.bitcast_convert_type(x, jnp.int32)
    sgn = jax.lax.shift_right_arithmetic(xi, 31)
    ukey = jax.lax.bitcast_convert_type(xi ^ (sgn | jnp.int32(-(2 ** 31))),
                                        jnp.uint32)

    def step(i, prefix):
        cand = prefix | (jnp.uint32(0x80000000) >> i)
        cnt = jnp.sum((ukey >= cand).astype(jnp.int32), axis=1, keepdims=True)
        return jnp.where(cnt >= _K, cand, prefix)

    t = jax.lax.fori_loop(
        0, 32, step, jnp.zeros((x.shape[0], 1), jnp.uint32))
    gt = ukey > t
    eq = ukey == t
    cnt_gt = jnp.sum(gt.astype(jnp.int32), axis=1, keepdims=True)
    r = _K - cnt_gt
    col = jax.lax.broadcasted_iota(jnp.int32, x.shape, 1)

    def istep(i, p):
        cand = p + (jnp.int32(16384) >> i)
        below = jnp.sum((eq & (col < cand)).astype(jnp.int32),
                        axis=1, keepdims=True)
        return jnp.where(below < r, cand, p)

    istar = jax.lax.fori_loop(
        0, 15, istep, jnp.zeros((x.shape[0], 1), jnp.int32))
    mask = gt | (eq & (col <= istar))
    o_ref[...] = mask.astype(jnp.int8)


def kernel(logits):
    n_rows, n_cols = logits.shape
    grid = (n_rows // _ROWS_PER_BLOCK,)
    out = pl.pallas_call(
        _mask_body,
        grid=grid,
        in_specs=[pl.BlockSpec((_ROWS_PER_BLOCK, n_cols), lambda i: (i, 0))],
        out_specs=pl.BlockSpec((_ROWS_PER_BLOCK, n_cols), lambda i: (i, 0)),
        out_shape=jax.ShapeDtypeStruct((n_rows, n_cols), jnp.int8),
    )(logits)
    return out.astype(jnp.bool_)
